# baseline (speedup 1.0000x reference)
import jax
import jax.numpy as jnp
from jax import lax
from jax.experimental import pallas as pl
from jax.experimental.pallas import tpu as pltpu
from jax.experimental.pallas import tpu_sc as plsc

NC, NS, LANES = 2, 16, 16
NW = NC * NS

N, C, E, G = 10000, 128, 320000, 64
BN = 2048
GRID = (N + BN - 1) // BN

COLS = 1024
NPAD = NS * COLS
PADNODE = NPAD - 1

EW = E // NW
CH_W = EW // 16
ROWS_W = CH_W // 8 + 1
BUF_W = (EW // 128 + 2) * 128
FIRE = 8


def _h_body(x_ref, wg_ref, h_ref):
    i = pl.program_id(0)
    hv = jnp.dot(x_ref[...], wg_ref[...], preferred_element_type=jnp.float32)
    h_ref[pl.ds(i * BN, BN)] = hv.reshape(BN)


def _run_h(x, Wg):
    return pl.pallas_call(
        _h_body,
        grid=(GRID,),
        in_specs=[
            pl.BlockSpec((BN, C), lambda i: (i, 0)),
            pl.BlockSpec((C, 1), lambda i: (0, 0)),
        ],
        out_specs=pl.BlockSpec((NPAD,), lambda i: (0,)),
        out_shape=jax.ShapeDtypeStruct((NPAD,), jnp.float32),
    )(x, Wg)


def _stats_body(x_ref, b_ref, w1_ref, b1_ref, w2_ref, b2_ref,
                chn_ref, sums_ref, cnt_ref):
    i = pl.program_id(0)
    xb = x_ref[...]
    bi = b_ref[...].reshape(BN, 1)
    hit = bi == lax.broadcasted_iota(jnp.int32, (BN, G), 1)

    @pl.when(i == 0)
    def _():
        sums_ref[...] = jnp.zeros_like(sums_ref)
        cnt_ref[...] = jnp.zeros_like(cnt_ref)

    def accum(onehot, xv):
        psums = lax.dot_general(onehot, xv, (((0,), (0,)), ((), ())),
                                preferred_element_type=jnp.float32)
        pcnt = lax.dot_general(onehot, jnp.ones((BN, 1), jnp.float32),
                               (((0,), (0,)), ((), ())),
                               preferred_element_type=jnp.float32)
        sums_ref[...] += psums
        cnt_ref[...] += pcnt

    @pl.when(i < GRID - 1)
    def _():
        accum(jnp.where(hit, 1.0, 0.0), xb)

    @pl.when(i == GRID - 1)
    def _():
        rid = lax.broadcasted_iota(jnp.int32, (BN, 1), 0) + i * BN
        valid = rid < N
        accum(jnp.where(valid & hit, 1.0, 0.0), jnp.where(valid, xb, 0.0))

    @pl.when(i == GRID - 1)
    def _():
        means = sums_ref[...] / jnp.maximum(cnt_ref[...], 1.0)
        t = jnp.maximum(
            jnp.dot(means, w1_ref[...], preferred_element_type=jnp.float32)
            + b1_ref[...].reshape(1, G), 0.0)
        chn_ref[...] = jax.nn.sigmoid(
            jnp.dot(t, w2_ref[...], preferred_element_type=jnp.float32)
            + b2_ref[...].reshape(1, C))


def _run_stats(x, batch, W1, b1, W2, b2):
    return pl.pallas_call(
        _stats_body,
        grid=(GRID,),
        in_specs=[
            pl.BlockSpec((BN, C), lambda i: (i, 0)),
            pl.BlockSpec((BN,), lambda i: (i,)),
            pl.BlockSpec((C, G), lambda i: (0, 0)),
            pl.BlockSpec((G,), lambda i: (0,)),
            pl.BlockSpec((G, C), lambda i: (0, 0)),
            pl.BlockSpec((C,), lambda i: (0,)),
        ],
        out_specs=pl.BlockSpec((G, C), lambda i: (0, 0)),
        out_shape=jax.ShapeDtypeStruct((G, C), jnp.float32),
        scratch_shapes=[
            pltpu.VMEM((G, C), jnp.float32),
            pltpu.VMEM((G, 1), jnp.float32),
        ],
    )(x, batch, W1, b1, W2, b2)


def _rsqrt3(d):
    yi = plsc.bitcast(d, jnp.int32)
    yi = 0x5F3759DF - lax.shift_right_arithmetic(yi, 1)
    y = plsc.bitcast(yi, jnp.float32)
    half = 0.5 * d
    y = y * (1.5 - half * y * y)
    y = y * (1.5 - half * y * y)
    y = y * (1.5 - half * y * y)
    return y


def _pipe_scatter(vals_row, idx2d, target, sem, build_chunk):
    prev = []
    for base in range(0, ROWS_W, FIRE):
        k = min(FIRE, ROWS_W - base)
        build_chunk(base, k)
        cur = [
            pltpu.async_copy(vals_row(base + j),
                             target.at[idx2d.at[base + j]], sem, add=True)
            for j in range(k)
        ]
        for dsc in prev:
            dsc.wait()
        prev = cur
    for dsc in prev:
        dsc.wait()


def _edge_span(wid):
    pre = (wid * EW) % 128
    ab = pl.multiple_of(wid * EW - pre, 128)
    return pre, ab


def _start_edge_load(ei, ab, wid, eb_v, sem):
    @pl.when(wid < NW - 1)
    def _():
        pltpu.async_copy(ei.at[:, pl.ds(ab, BUF_W)], eb_v, sem)

    @pl.when(wid == NW - 1)
    def _():
        pltpu.async_copy(ei.at[:, pl.ds(ab, BUF_W - 128)],
                         eb_v.at[:, pl.ds(0, BUF_W - 128)], sem)


def _wait_edge_load(ei, ab, wid, eb_v, sem):
    @pl.when(wid < NW - 1)
    def _():
        pltpu.make_async_copy(ei.at[:, pl.ds(ab, BUF_W)], eb_v, sem).wait()

    @pl.when(wid == NW - 1)
    def _():
        pltpu.make_async_copy(ei.at[:, pl.ds(ab, BUF_W - 128)],
                              eb_v.at[:, pl.ds(0, BUF_W - 128)], sem).wait()


def _fill_zeros(zeros_v):
    @plsc.parallel_loop(0, COLS // 16, unroll=4)
    def _(j):
        zeros_v[pl.ds(j * 16, 16)] = jnp.zeros((16,), jnp.float32)


def _deg_body(ei, degp, zeros_v, ones_v, eb_v, dst2_v, deg_s, sem, semw):
    c = lax.axis_index("c")
    s = lax.axis_index("s")
    wid = c * NS + s
    padv = jnp.full((16,), PADNODE, jnp.int32)
    pre, ab = _edge_span(wid)

    _start_edge_load(ei, ab, wid, eb_v, semw)
    _fill_zeros(zeros_v)
    for k in range(128 // 16):
        ones_v[pl.ds(k * 16, 16)] = jnp.ones((16,), jnp.float32)
    pltpu.sync_copy(zeros_v, deg_s.at[pl.ds(s * COLS, COLS)])
    plsc.subcore_barrier()

    _wait_edge_load(ei, ab, wid, eb_v, semw)

    def build(base, k):
        @plsc.parallel_loop(base * 8, (base + k) * 8, unroll=4)
        def _(j):
            off = jnp.minimum(pre + j * 16, BUF_W - 16)
            v = jnp.where(j < CH_W, eb_v[1, pl.ds(off, 16)], padv)
            dst2_v[j >> 3, pl.ds((j & 7) * 16, 16)] = v

    _pipe_scatter(lambda j: ones_v, dst2_v, deg_s, sem, build)
    plsc.subcore_barrier()

    pltpu.sync_copy(deg_s.at[pl.ds(s * COLS, COLS)],
                    degp.at[c, pl.ds(s * COLS, COLS)])


def _run_deg(ei):
    mesh = plsc.VectorSubcoreMesh(core_axis_name="c", subcore_axis_name="s",
                                  num_cores=NC, num_subcores=NS)
    return pl.kernel(
        _deg_body,
        out_type=jax.ShapeDtypeStruct((NC, NPAD), jnp.float32),
        mesh=mesh,
        compiler_params=pltpu.CompilerParams(needs_layout_passes=False),
        scratch_types=[
            pltpu.VMEM((COLS,), jnp.float32),
            pltpu.VMEM((128,), jnp.float32),
            pltpu.VMEM((2, BUF_W), jnp.int32),
            pltpu.VMEM((ROWS_W, 128), jnp.int32),
            pltpu.VMEM_SHARED((NPAD,), jnp.float32),
            pltpu.SemaphoreType.DMA,
            pltpu.SemaphoreType.DMA,
        ],
    )(ei)


def _main_body(ei, h1d, degp, acc0_out, acc1_out, dinv_out,
               zeros_v, eb_v, dst2_v, qv_v, q_v, d_v, d2_v, h_v,
               q_s, acc_s, sem, semw, semh):
    c = lax.axis_index("c")
    s = lax.axis_index("s")
    wid = c * NS + s
    padv = jnp.full((16,), PADNODE, jnp.int32)
    pre, ab = _edge_span(wid)

    _start_edge_load(ei, ab, wid, eb_v, semw)
    pltpu.async_copy(h1d.at[pl.ds(s * COLS, COLS)], h_v, semh)
    _fill_zeros(zeros_v)
    pltpu.sync_copy(zeros_v, acc_s.at[pl.ds(s * COLS, COLS)])
    pltpu.sync_copy(degp.at[0, pl.ds(s * COLS, COLS)], d_v)
    pltpu.sync_copy(degp.at[1, pl.ds(s * COLS, COLS)], d2_v)
    plsc.subcore_barrier()

    pltpu.make_async_copy(h1d.at[pl.ds(0, COLS)], h_v, semh).wait()

    @plsc.parallel_loop(0, COLS // 16, unroll=4)
    def _(j):
        sl = pl.ds(j * 16, 16)
        y = _rsqrt3(d_v[sl] + d2_v[sl] + 1.0)
        d_v[sl] = y
        h_v[sl] = y * h_v[sl]
    pltpu.sync_copy(h_v, q_s.at[pl.ds(s * COLS, COLS)])

    @pl.when(c == 0)
    def _():
        pltpu.sync_copy(d_v, dinv_out.at[pl.ds(s * COLS, COLS)])

    plsc.subcore_barrier()

    pltpu.sync_copy(q_s, q_v)
    _wait_edge_load(ei, ab, wid, eb_v, semw)

    def build(base, k):
        @plsc.parallel_loop(base * 8, (base + k) * 8, unroll=4)
        def _(j):
            off = jnp.minimum(pre + j * 16, BUF_W - 16)
            sl16 = pl.ds((j & 7) * 16, 16)
            row = j >> 3
            sidx = jnp.where(j < CH_W, eb_v[0, pl.ds(off, 16)], padv)
            didx = jnp.where(j < CH_W, eb_v[1, pl.ds(off, 16)], padv)
            qv_v[row, sl16] = plsc.load_gather(q_v, [sidx])
            dst2_v[row, sl16] = didx

    _pipe_scatter(lambda j: qv_v.at[j], dst2_v, acc_s, sem, build)
    plsc.subcore_barrier()

    @pl.when(c == 0)
    def _():
        pltpu.sync_copy(acc_s.at[pl.ds(s * COLS, COLS)],
                        acc0_out.at[pl.ds(s * COLS, COLS)])

    @pl.when(c == 1)
    def _():
        pltpu.sync_copy(acc_s.at[pl.ds(s * COLS, COLS)],
                        acc1_out.at[pl.ds(s * COLS, COLS)])


def _run_main(ei, h1d, degp):
    mesh = plsc.VectorSubcoreMesh(core_axis_name="c", subcore_axis_name="s",
                                  num_cores=NC, num_subcores=NS)
    return pl.kernel(
        _main_body,
        out_type=(
            jax.ShapeDtypeStruct((NPAD,), jnp.float32),
            jax.ShapeDtypeStruct((NPAD,), jnp.float32),
            jax.ShapeDtypeStruct((NPAD,), jnp.float32),
        ),
        mesh=mesh,
        compiler_params=pltpu.CompilerParams(needs_layout_passes=False),
        scratch_types=[
            pltpu.VMEM((COLS,), jnp.float32),
            pltpu.VMEM((2, BUF_W), jnp.int32),
            pltpu.VMEM((ROWS_W, 128), jnp.int32),
            pltpu.VMEM((ROWS_W, 128), jnp.float32),
            pltpu.VMEM((NPAD,), jnp.float32),
            pltpu.VMEM((COLS,), jnp.float32),
            pltpu.VMEM((COLS,), jnp.float32),
            pltpu.VMEM((COLS,), jnp.float32),
            pltpu.VMEM_SHARED((NPAD,), jnp.float32),
            pltpu.VMEM_SHARED((NPAD,), jnp.float32),
            pltpu.SemaphoreType.DMA,
            pltpu.SemaphoreType.DMA,
            pltpu.SemaphoreType.DMA,
        ],
    )(ei, h1d, degp)


def _combine_body(x_ref, b_ref, chn_ref, dinv_ref, acc0_ref, acc1_ref,
                  h1_ref, bg_ref, o_ref):
    i = pl.program_id(0)
    xb = x_ref[...]
    bi = b_ref[...].reshape(BN, 1)
    onehot = (bi == lax.broadcasted_iota(jnp.int32, (BN, G), 1)
              ).astype(jnp.float32)
    chn_rows = jnp.dot(onehot, chn_ref[...], preferred_element_type=jnp.float32)
    sl = pl.ds(i * BN, BN)
    dinv = dinv_ref[sl]
    gcn = dinv * (acc0_ref[sl] + acc1_ref[sl]) + dinv * dinv * h1_ref[sl]
    spa = jax.nn.sigmoid(gcn + bg_ref[...]).reshape(BN, 1)
    o_ref[...] = xb * (1.0 + spa + chn_rows)


def _run_combine(x, batch, chn, dinv, acc0, acc1, h, bg):
    full = pl.BlockSpec((NPAD,), lambda i: (0,))
    return pl.pallas_call(
        _combine_body,
        grid=(GRID,),
        in_specs=[
            pl.BlockSpec((BN, C), lambda i: (i, 0)),
            pl.BlockSpec((BN,), lambda i: (i,)),
            pl.BlockSpec((G, C), lambda i: (0, 0)),
            full, full, full, full,
            pl.BlockSpec((1,), lambda i: (0,)),
        ],
        out_specs=pl.BlockSpec((BN, C), lambda i: (i, 0)),
        out_shape=jax.ShapeDtypeStruct((N, C), jnp.float32),
    )(x, batch, chn, dinv, acc0, acc1, h, bg)


@jax.jit
def kernel(x, batch, edge_index, W1, b1, W2, b2, Wg, bg):
    h = _run_h(x, Wg)
    degp = _run_deg(edge_index)
    chn = _run_stats(x, batch, W1, b1, W2, b2)
    acc0, acc1, dinv = _run_main(edge_index, h, degp)
    return _run_combine(x, batch, chn, dinv, acc0, acc1, h, bg)


# speedup vs baseline: 1.0218x; 1.0218x over previous
import jax
import jax.numpy as jnp
from jax import lax
from jax.experimental import pallas as pl
from jax.experimental.pallas import tpu as pltpu
from jax.experimental.pallas import tpu_sc as plsc

NC, NS, LANES = 2, 16, 16
NW = NC * NS

N, C, E, G = 10000, 128, 320000, 64
BN = 2048
GRID = (N + BN - 1) // BN

COLS = 1024
NPAD = NS * COLS
PADNODE = NPAD - 1

EW = E // NW
CH_W = EW // 16
ROWS_W = CH_W // 8 + 1
BUF_W = (EW // 128 + 2) * 128
FIRE = 16


def _h_body(x_ref, wg_ref, h_ref):
    i = pl.program_id(0)
    hv = jnp.dot(x_ref[...], wg_ref[...], preferred_element_type=jnp.float32)
    h_ref[pl.ds(i * BN, BN)] = hv.reshape(BN)


def _run_h(x, Wg):
    return pl.pallas_call(
        _h_body,
        grid=(GRID,),
        in_specs=[
            pl.BlockSpec((BN, C), lambda i: (i, 0)),
            pl.BlockSpec((C, 1), lambda i: (0, 0)),
        ],
        out_specs=pl.BlockSpec((NPAD,), lambda i: (0,)),
        out_shape=jax.ShapeDtypeStruct((NPAD,), jnp.float32),
    )(x, Wg)


def _stats_body(x_ref, b_ref, w1_ref, b1_ref, w2_ref, b2_ref,
                chn_ref, sums_ref, cnt_ref):
    i = pl.program_id(0)
    xb = x_ref[...]
    bi = b_ref[...].reshape(BN, 1)
    hit = bi == lax.broadcasted_iota(jnp.int32, (BN, G), 1)

    @pl.when(i == 0)
    def _():
        sums_ref[...] = jnp.zeros_like(sums_ref)
        cnt_ref[...] = jnp.zeros_like(cnt_ref)

    def accum(onehot, xv):
        psums = lax.dot_general(onehot, xv, (((0,), (0,)), ((), ())),
                                preferred_element_type=jnp.float32)
        pcnt = lax.dot_general(onehot, jnp.ones((BN, 1), jnp.float32),
                               (((0,), (0,)), ((), ())),
                               preferred_element_type=jnp.float32)
        sums_ref[...] += psums
        cnt_ref[...] += pcnt

    @pl.when(i < GRID - 1)
    def _():
        accum(jnp.where(hit, 1.0, 0.0), xb)

    @pl.when(i == GRID - 1)
    def _():
        rid = lax.broadcasted_iota(jnp.int32, (BN, 1), 0) + i * BN
        valid = rid < N
        accum(jnp.where(valid & hit, 1.0, 0.0), jnp.where(valid, xb, 0.0))

    @pl.when(i == GRID - 1)
    def _():
        means = sums_ref[...] / jnp.maximum(cnt_ref[...], 1.0)
        t = jnp.maximum(
            jnp.dot(means, w1_ref[...], preferred_element_type=jnp.float32)
            + b1_ref[...].reshape(1, G), 0.0)
        chn_ref[...] = jax.nn.sigmoid(
            jnp.dot(t, w2_ref[...], preferred_element_type=jnp.float32)
            + b2_ref[...].reshape(1, C))


def _run_stats(x, batch, W1, b1, W2, b2):
    return pl.pallas_call(
        _stats_body,
        grid=(GRID,),
        in_specs=[
            pl.BlockSpec((BN, C), lambda i: (i, 0)),
            pl.BlockSpec((BN,), lambda i: (i,)),
            pl.BlockSpec((C, G), lambda i: (0, 0)),
            pl.BlockSpec((G,), lambda i: (0,)),
            pl.BlockSpec((G, C), lambda i: (0, 0)),
            pl.BlockSpec((C,), lambda i: (0,)),
        ],
        out_specs=pl.BlockSpec((G, C), lambda i: (0, 0)),
        out_shape=jax.ShapeDtypeStruct((G, C), jnp.float32),
        scratch_shapes=[
            pltpu.VMEM((G, C), jnp.float32),
            pltpu.VMEM((G, 1), jnp.float32),
        ],
    )(x, batch, W1, b1, W2, b2)


def _rsqrt3(d):
    yi = plsc.bitcast(d, jnp.int32)
    yi = 0x5F3759DF - lax.shift_right_arithmetic(yi, 1)
    y = plsc.bitcast(yi, jnp.float32)
    half = 0.5 * d
    y = y * (1.5 - half * y * y)
    y = y * (1.5 - half * y * y)
    y = y * (1.5 - half * y * y)
    return y


def _pipe_scatter(vals_row, idx2d, target, sem, build_chunk):
    prev = []
    for base in range(0, ROWS_W, FIRE):
        k = min(FIRE, ROWS_W - base)
        build_chunk(base, k)
        cur = [
            pltpu.async_copy(vals_row(base + j),
                             target.at[idx2d.at[base + j]], sem, add=True)
            for j in range(k)
        ]
        for dsc in prev:
            dsc.wait()
        prev = cur
    for dsc in prev:
        dsc.wait()


def _edge_span(wid):
    pre = (wid * EW) % 128
    ab = pl.multiple_of(wid * EW - pre, 128)
    return pre, ab


def _start_edge_load(ei, ab, wid, eb_v, sem):
    @pl.when(wid < NW - 1)
    def _():
        pltpu.async_copy(ei.at[:, pl.ds(ab, BUF_W)], eb_v, sem)

    @pl.when(wid == NW - 1)
    def _():
        pltpu.async_copy(ei.at[:, pl.ds(ab, BUF_W - 128)],
                         eb_v.at[:, pl.ds(0, BUF_W - 128)], sem)


def _wait_edge_load(ei, ab, wid, eb_v, sem):
    @pl.when(wid < NW - 1)
    def _():
        pltpu.make_async_copy(ei.at[:, pl.ds(ab, BUF_W)], eb_v, sem).wait()

    @pl.when(wid == NW - 1)
    def _():
        pltpu.make_async_copy(ei.at[:, pl.ds(ab, BUF_W - 128)],
                              eb_v.at[:, pl.ds(0, BUF_W - 128)], sem).wait()


def _fill_zeros(zeros_v):
    @plsc.parallel_loop(0, COLS // 16, unroll=4)
    def _(j):
        zeros_v[pl.ds(j * 16, 16)] = jnp.zeros((16,), jnp.float32)


def _deg_body(ei, degp, zeros_v, ones_v, eb_v, dst2_v, deg_s, sem, semw):
    c = lax.axis_index("c")
    s = lax.axis_index("s")
    wid = c * NS + s
    padv = jnp.full((16,), PADNODE, jnp.int32)
    pre, ab = _edge_span(wid)

    _start_edge_load(ei, ab, wid, eb_v, semw)
    _fill_zeros(zeros_v)
    for k in range(128 // 16):
        ones_v[pl.ds(k * 16, 16)] = jnp.ones((16,), jnp.float32)
    pltpu.sync_copy(zeros_v, deg_s.at[pl.ds(s * COLS, COLS)])
    plsc.subcore_barrier()

    _wait_edge_load(ei, ab, wid, eb_v, semw)

    def build(base, k):
        @plsc.parallel_loop(base * 8, (base + k) * 8, unroll=4)
        def _(j):
            off = jnp.minimum(pre + j * 16, BUF_W - 16)
            v = jnp.where(j < CH_W, eb_v[1, pl.ds(off, 16)], padv)
            dst2_v[j >> 3, pl.ds((j & 7) * 16, 16)] = v

    _pipe_scatter(lambda j: ones_v, dst2_v, deg_s, sem, build)
    plsc.subcore_barrier()

    pltpu.sync_copy(deg_s.at[pl.ds(s * COLS, COLS)],
                    degp.at[c, pl.ds(s * COLS, COLS)])


def _run_deg(ei):
    mesh = plsc.VectorSubcoreMesh(core_axis_name="c", subcore_axis_name="s",
                                  num_cores=NC, num_subcores=NS)
    return pl.kernel(
        _deg_body,
        out_type=jax.ShapeDtypeStruct((NC, NPAD), jnp.float32),
        mesh=mesh,
        compiler_params=pltpu.CompilerParams(needs_layout_passes=False),
        scratch_types=[
            pltpu.VMEM((COLS,), jnp.float32),
            pltpu.VMEM((128,), jnp.float32),
            pltpu.VMEM((2, BUF_W), jnp.int32),
            pltpu.VMEM((ROWS_W, 128), jnp.int32),
            pltpu.VMEM_SHARED((NPAD,), jnp.float32),
            pltpu.SemaphoreType.DMA,
            pltpu.SemaphoreType.DMA,
        ],
    )(ei)


def _main_body(ei, h1d, degp, acc0_out, acc1_out, dinv_out,
               zeros_v, eb_v, dst2_v, qv_v, q_v, d_v, d2_v, h_v,
               q_s, acc_s, sem, semw, semh):
    c = lax.axis_index("c")
    s = lax.axis_index("s")
    wid = c * NS + s
    padv = jnp.full((16,), PADNODE, jnp.int32)
    pre, ab = _edge_span(wid)

    _start_edge_load(ei, ab, wid, eb_v, semw)
    pltpu.async_copy(h1d.at[pl.ds(s * COLS, COLS)], h_v, semh)
    _fill_zeros(zeros_v)
    pltpu.sync_copy(zeros_v, acc_s.at[pl.ds(s * COLS, COLS)])
    pltpu.sync_copy(degp.at[0, pl.ds(s * COLS, COLS)], d_v)
    pltpu.sync_copy(degp.at[1, pl.ds(s * COLS, COLS)], d2_v)
    plsc.subcore_barrier()

    pltpu.make_async_copy(h1d.at[pl.ds(0, COLS)], h_v, semh).wait()

    @plsc.parallel_loop(0, COLS // 16, unroll=4)
    def _(j):
        sl = pl.ds(j * 16, 16)
        y = _rsqrt3(d_v[sl] + d2_v[sl] + 1.0)
        d_v[sl] = y
        h_v[sl] = y * h_v[sl]
    pltpu.sync_copy(h_v, q_s.at[pl.ds(s * COLS, COLS)])

    @pl.when(c == 0)
    def _():
        pltpu.sync_copy(d_v, dinv_out.at[pl.ds(s * COLS, COLS)])

    plsc.subcore_barrier()

    pltpu.sync_copy(q_s, q_v)
    _wait_edge_load(ei, ab, wid, eb_v, semw)

    def build(base, k):
        @plsc.parallel_loop(base * 8, (base + k) * 8, unroll=4)
        def _(j):
            off = jnp.minimum(pre + j * 16, BUF_W - 16)
            sl16 = pl.ds((j & 7) * 16, 16)
            row = j >> 3
            sidx = jnp.where(j < CH_W, eb_v[0, pl.ds(off, 16)], padv)
            didx = jnp.where(j < CH_W, eb_v[1, pl.ds(off, 16)], padv)
            qv_v[row, sl16] = plsc.load_gather(q_v, [sidx])
            dst2_v[row, sl16] = didx

    _pipe_scatter(lambda j: qv_v.at[j], dst2_v, acc_s, sem, build)
    plsc.subcore_barrier()

    @pl.when(c == 0)
    def _():
        pltpu.sync_copy(acc_s.at[pl.ds(s * COLS, COLS)],
                        acc0_out.at[pl.ds(s * COLS, COLS)])

    @pl.when(c == 1)
    def _():
        pltpu.sync_copy(acc_s.at[pl.ds(s * COLS, COLS)],
                        acc1_out.at[pl.ds(s * COLS, COLS)])


def _run_main(ei, h1d, degp):
    mesh = plsc.VectorSubcoreMesh(core_axis_name="c", subcore_axis_name="s",
                                  num_cores=NC, num_subcores=NS)
    return pl.kernel(
        _main_body,
        out_type=(
            jax.ShapeDtypeStruct((NPAD,), jnp.float32),
            jax.ShapeDtypeStruct((NPAD,), jnp.float32),
            jax.ShapeDtypeStruct((NPAD,), jnp.float32),
        ),
        mesh=mesh,
        compiler_params=pltpu.CompilerParams(needs_layout_passes=False),
        scratch_types=[
            pltpu.VMEM((COLS,), jnp.float32),
            pltpu.VMEM((2, BUF_W), jnp.int32),
            pltpu.VMEM((ROWS_W, 128), jnp.int32),
            pltpu.VMEM((ROWS_W, 128), jnp.float32),
            pltpu.VMEM((NPAD,), jnp.float32),
            pltpu.VMEM((COLS,), jnp.float32),
            pltpu.VMEM((COLS,), jnp.float32),
            pltpu.VMEM((COLS,), jnp.float32),
            pltpu.VMEM_SHARED((NPAD,), jnp.float32),
            pltpu.VMEM_SHARED((NPAD,), jnp.float32),
            pltpu.SemaphoreType.DMA,
            pltpu.SemaphoreType.DMA,
            pltpu.SemaphoreType.DMA,
        ],
    )(ei, h1d, degp)


def _combine_body(x_ref, b_ref, chn_ref, dinv_ref, acc0_ref, acc1_ref,
                  h1_ref, bg_ref, o_ref):
    i = pl.program_id(0)
    xb = x_ref[...]
    bi = b_ref[...].reshape(BN, 1)
    onehot = (bi == lax.broadcasted_iota(jnp.int32, (BN, G), 1)
              ).astype(jnp.float32)
    chn_rows = jnp.dot(onehot, chn_ref[...], preferred_element_type=jnp.float32)
    sl = pl.ds(i * BN, BN)
    dinv = dinv_ref[sl]
    gcn = dinv * (acc0_ref[sl] + acc1_ref[sl]) + dinv * dinv * h1_ref[sl]
    spa = jax.nn.sigmoid(gcn + bg_ref[...]).reshape(BN, 1)
    o_ref[...] = xb * (1.0 + spa + chn_rows)


def _run_combine(x, batch, chn, dinv, acc0, acc1, h, bg):
    full = pl.BlockSpec((NPAD,), lambda i: (0,))
    return pl.pallas_call(
        _combine_body,
        grid=(GRID,),
        in_specs=[
            pl.BlockSpec((BN, C), lambda i: (i, 0)),
            pl.BlockSpec((BN,), lambda i: (i,)),
            pl.BlockSpec((G, C), lambda i: (0, 0)),
            full, full, full, full,
            pl.BlockSpec((1,), lambda i: (0,)),
        ],
        out_specs=pl.BlockSpec((BN, C), lambda i: (i, 0)),
        out_shape=jax.ShapeDtypeStruct((N, C), jnp.float32),
    )(x, batch, chn, dinv, acc0, acc1, h, bg)


@jax.jit
def kernel(x, batch, edge_index, W1, b1, W2, b2, Wg, bg):
    h = _run_h(x, Wg)
    degp = _run_deg(edge_index)
    chn = _run_stats(x, batch, W1, b1, W2, b2)
    acc0, acc1, dinv = _run_main(edge_index, h, degp)
    return _run_combine(x, batch, chn, dinv, acc0, acc1, h, bg)


# speedup vs baseline: 1.0396x; 1.0174x over previous
import jax
import jax.numpy as jnp
from jax import lax
from jax.experimental import pallas as pl
from jax.experimental.pallas import tpu as pltpu
from jax.experimental.pallas import tpu_sc as plsc

NC, NS, LANES = 2, 16, 16
NW = NC * NS

N, C, E, G = 10000, 128, 320000, 64
BN = 2048
GRID = (N + BN - 1) // BN

COLS = 1024
NPAD = NS * COLS
PADNODE = NPAD - 1

EW = E // NW
CH_W = EW // 16
ROWS_W = CH_W // 8 + 1
BUF_W = (EW // 128 + 2) * 128
FIRE = 16


def _stats_body(x_ref, b_ref, w1_ref, b1_ref, w2_ref, b2_ref, wg_ref,
                h_ref, chn_ref, sums_ref, cnt_ref):
    i = pl.program_id(0)
    xb = x_ref[...]
    bi = b_ref[...].reshape(BN, 1)
    hit = bi == lax.broadcasted_iota(jnp.int32, (BN, G), 1)

    @pl.when(i == 0)
    def _():
        sums_ref[...] = jnp.zeros_like(sums_ref)
        cnt_ref[...] = jnp.zeros_like(cnt_ref)

    def accum(onehot, xv):
        psums = lax.dot_general(onehot, xv, (((0,), (0,)), ((), ())),
                                preferred_element_type=jnp.float32)
        pcnt = lax.dot_general(onehot, jnp.ones((BN, 1), jnp.float32),
                               (((0,), (0,)), ((), ())),
                               preferred_element_type=jnp.float32)
        sums_ref[...] += psums
        cnt_ref[...] += pcnt

    @pl.when(i < GRID - 1)
    def _():
        accum(jnp.where(hit, 1.0, 0.0), xb)

    @pl.when(i == GRID - 1)
    def _():
        rid = lax.broadcasted_iota(jnp.int32, (BN, 1), 0) + i * BN
        valid = rid < N
        accum(jnp.where(valid & hit, 1.0, 0.0), jnp.where(valid, xb, 0.0))

    hv = jnp.dot(xb, wg_ref[...], preferred_element_type=jnp.float32)
    h_ref[pl.ds(i * BN, BN)] = hv.reshape(BN)

    @pl.when(i == GRID - 1)
    def _():
        means = sums_ref[...] / jnp.maximum(cnt_ref[...], 1.0)
        t = jnp.maximum(
            jnp.dot(means, w1_ref[...], preferred_element_type=jnp.float32)
            + b1_ref[...].reshape(1, G), 0.0)
        chn_ref[...] = jax.nn.sigmoid(
            jnp.dot(t, w2_ref[...], preferred_element_type=jnp.float32)
            + b2_ref[...].reshape(1, C))


def _run_stats(x, batch, W1, b1, W2, b2, Wg):
    return pl.pallas_call(
        _stats_body,
        grid=(GRID,),
        in_specs=[
            pl.BlockSpec((BN, C), lambda i: (i, 0)),
            pl.BlockSpec((BN,), lambda i: (i,)),
            pl.BlockSpec((C, G), lambda i: (0, 0)),
            pl.BlockSpec((G,), lambda i: (0,)),
            pl.BlockSpec((G, C), lambda i: (0, 0)),
            pl.BlockSpec((C,), lambda i: (0,)),
            pl.BlockSpec((C, 1), lambda i: (0, 0)),
        ],
        out_specs=[
            pl.BlockSpec((NPAD,), lambda i: (0,)),
            pl.BlockSpec((G, C), lambda i: (0, 0)),
        ],
        out_shape=[
            jax.ShapeDtypeStruct((NPAD,), jnp.float32),
            jax.ShapeDtypeStruct((G, C), jnp.float32),
        ],
        scratch_shapes=[
            pltpu.VMEM((G, C), jnp.float32),
            pltpu.VMEM((G, 1), jnp.float32),
        ],
    )(x, batch, W1, b1, W2, b2, Wg)


def _rsqrt3(d):
    yi = plsc.bitcast(d, jnp.int32)
    yi = 0x5F3759DF - lax.shift_right_arithmetic(yi, 1)
    y = plsc.bitcast(yi, jnp.float32)
    half = 0.5 * d
    y = y * (1.5 - half * y * y)
    y = y * (1.5 - half * y * y)
    y = y * (1.5 - half * y * y)
    return y


def _pipe_scatter(vals_row, idx2d, target, sem, build_chunk, wait_half2):
    prev = []
    for base in range(0, ROWS_W, FIRE):
        if base == 32:
            wait_half2()
        k = min(FIRE, ROWS_W - base)
        build_chunk(base, k)
        cur = [
            pltpu.async_copy(vals_row(base + j),
                             target.at[idx2d.at[base + j]], sem, add=True)
            for j in range(k)
        ]
        for dsc in prev:
            dsc.wait()
        prev = cur
    for dsc in prev:
        dsc.wait()


def _edge_span(wid):
    pre = (wid * EW) % 128
    ab = pl.multiple_of(wid * EW - pre, 128)
    return pre, ab


HALF1 = 5248
HALF2 = BUF_W - HALF1


def _start_edge_load(ei, ab, wid, eb_v, sem):
    pltpu.async_copy(ei.at[:, pl.ds(ab, HALF1)],
                     eb_v.at[:, pl.ds(0, HALF1)], sem)

    @pl.when(wid < NW - 1)
    def _():
        pltpu.async_copy(ei.at[:, pl.ds(ab + HALF1, HALF2)],
                         eb_v.at[:, pl.ds(HALF1, HALF2)], sem)

    @pl.when(wid == NW - 1)
    def _():
        pltpu.async_copy(ei.at[:, pl.ds(ab + HALF1, HALF2 - 128)],
                         eb_v.at[:, pl.ds(HALF1, HALF2 - 128)], sem)


def _wait_edge_half1(ei, ab, eb_v, sem):
    pltpu.make_async_copy(ei.at[:, pl.ds(ab, HALF1)],
                          eb_v.at[:, pl.ds(0, HALF1)], sem).wait()


def _wait_edge_half2(ei, ab, wid, eb_v, sem):
    @pl.when(wid < NW - 1)
    def _():
        pltpu.make_async_copy(ei.at[:, pl.ds(ab + HALF1, HALF2)],
                              eb_v.at[:, pl.ds(HALF1, HALF2)], sem).wait()

    @pl.when(wid == NW - 1)
    def _():
        pltpu.make_async_copy(ei.at[:, pl.ds(ab + HALF1, HALF2 - 128)],
                              eb_v.at[:, pl.ds(HALF1, HALF2 - 128)],
                              sem).wait()


def _fill_zeros(zeros_v):
    @plsc.parallel_loop(0, COLS // 16, unroll=4)
    def _(j):
        zeros_v[pl.ds(j * 16, 16)] = jnp.zeros((16,), jnp.float32)


def _deg_body(ei, degp, zeros_v, ones_v, eb_v, dst2_v, deg_s, sem, semw):
    c = lax.axis_index("c")
    s = lax.axis_index("s")
    wid = c * NS + s
    padv = jnp.full((16,), PADNODE, jnp.int32)
    pre, ab = _edge_span(wid)

    _start_edge_load(ei, ab, wid, eb_v, semw)
    _fill_zeros(zeros_v)
    for k in range(128 // 16):
        ones_v[pl.ds(k * 16, 16)] = jnp.ones((16,), jnp.float32)
    pltpu.sync_copy(zeros_v, deg_s.at[pl.ds(s * COLS, COLS)])
    plsc.subcore_barrier()

    _wait_edge_half1(ei, ab, eb_v, semw)

    def build(base, k):
        @plsc.parallel_loop(base * 8, (base + k) * 8, unroll=4)
        def _(j):
            off = jnp.minimum(pre + j * 16, BUF_W - 16)
            v = jnp.where(j < CH_W, eb_v[1, pl.ds(off, 16)], padv)
            dst2_v[j >> 3, pl.ds((j & 7) * 16, 16)] = v

    _pipe_scatter(lambda j: ones_v, dst2_v, deg_s, sem, build,
                  lambda: _wait_edge_half2(ei, ab, wid, eb_v, semw))
    plsc.subcore_barrier()

    pltpu.sync_copy(deg_s.at[pl.ds(s * COLS, COLS)],
                    degp.at[c, pl.ds(s * COLS, COLS)])


def _run_deg(ei):
    mesh = plsc.VectorSubcoreMesh(core_axis_name="c", subcore_axis_name="s",
                                  num_cores=NC, num_subcores=NS)
    return pl.kernel(
        _deg_body,
        out_type=jax.ShapeDtypeStruct((NC, NPAD), jnp.float32),
        mesh=mesh,
        compiler_params=pltpu.CompilerParams(needs_layout_passes=False),
        scratch_types=[
            pltpu.VMEM((COLS,), jnp.float32),
            pltpu.VMEM((128,), jnp.float32),
            pltpu.VMEM((2, BUF_W), jnp.int32),
            pltpu.VMEM((ROWS_W, 128), jnp.int32),
            pltpu.VMEM_SHARED((NPAD,), jnp.float32),
            pltpu.SemaphoreType.DMA,
            pltpu.SemaphoreType.DMA,
        ],
    )(ei)


def _main_body(ei, h1d, degp, acc0_out, acc1_out, dinv_out,
               zeros_v, eb_v, dst2_v, qv_v, q_v, d_v, d2_v, h_v,
               q_s, acc_s, sem, semw, semh):
    c = lax.axis_index("c")
    s = lax.axis_index("s")
    wid = c * NS + s
    padv = jnp.full((16,), PADNODE, jnp.int32)
    pre, ab = _edge_span(wid)

    _start_edge_load(ei, ab, wid, eb_v, semw)
    pltpu.async_copy(h1d.at[pl.ds(s * COLS, COLS)], h_v, semh)
    _fill_zeros(zeros_v)
    pltpu.sync_copy(zeros_v, acc_s.at[pl.ds(s * COLS, COLS)])
    pltpu.sync_copy(degp.at[0, pl.ds(s * COLS, COLS)], d_v)
    pltpu.sync_copy(degp.at[1, pl.ds(s * COLS, COLS)], d2_v)
    plsc.subcore_barrier()

    pltpu.make_async_copy(h1d.at[pl.ds(0, COLS)], h_v, semh).wait()

    @plsc.parallel_loop(0, COLS // 16, unroll=4)
    def _(j):
        sl = pl.ds(j * 16, 16)
        y = _rsqrt3(d_v[sl] + d2_v[sl] + 1.0)
        d_v[sl] = y
        h_v[sl] = y * h_v[sl]
    pltpu.sync_copy(h_v, q_s.at[pl.ds(s * COLS, COLS)])

    @pl.when(c == 0)
    def _():
        pltpu.sync_copy(d_v, dinv_out.at[pl.ds(s * COLS, COLS)])

    plsc.subcore_barrier()

    pltpu.sync_copy(q_s, q_v)
    _wait_edge_half1(ei, ab, eb_v, semw)

    def build(base, k):
        @plsc.parallel_loop(base * 8, (base + k) * 8, unroll=4)
        def _(j):
            off = jnp.minimum(pre + j * 16, BUF_W - 16)
            sl16 = pl.ds((j & 7) * 16, 16)
            row = j >> 3
            sidx = jnp.where(j < CH_W, eb_v[0, pl.ds(off, 16)], padv)
            didx = jnp.where(j < CH_W, eb_v[1, pl.ds(off, 16)], padv)
            qv_v[row, sl16] = plsc.load_gather(q_v, [sidx])
            dst2_v[row, sl16] = didx

    _pipe_scatter(lambda j: qv_v.at[j], dst2_v, acc_s, sem, build,
                  lambda: _wait_edge_half2(ei, ab, wid, eb_v, semw))
    plsc.subcore_barrier()

    @pl.when(c == 0)
    def _():
        pltpu.sync_copy(acc_s.at[pl.ds(s * COLS, COLS)],
                        acc0_out.at[pl.ds(s * COLS, COLS)])

    @pl.when(c == 1)
    def _():
        pltpu.sync_copy(acc_s.at[pl.ds(s * COLS, COLS)],
                        acc1_out.at[pl.ds(s * COLS, COLS)])


def _run_main(ei, h1d, degp):
    mesh = plsc.VectorSubcoreMesh(core_axis_name="c", subcore_axis_name="s",
                                  num_cores=NC, num_subcores=NS)
    return pl.kernel(
        _main_body,
        out_type=(
            jax.ShapeDtypeStruct((NPAD,), jnp.float32),
            jax.ShapeDtypeStruct((NPAD,), jnp.float32),
            jax.ShapeDtypeStruct((NPAD,), jnp.float32),
        ),
        mesh=mesh,
        compiler_params=pltpu.CompilerParams(needs_layout_passes=False),
        scratch_types=[
            pltpu.VMEM((COLS,), jnp.float32),
            pltpu.VMEM((2, BUF_W), jnp.int32),
            pltpu.VMEM((ROWS_W, 128), jnp.int32),
            pltpu.VMEM((ROWS_W, 128), jnp.float32),
            pltpu.VMEM((NPAD,), jnp.float32),
            pltpu.VMEM((COLS,), jnp.float32),
            pltpu.VMEM((COLS,), jnp.float32),
            pltpu.VMEM((COLS,), jnp.float32),
            pltpu.VMEM_SHARED((NPAD,), jnp.float32),
            pltpu.VMEM_SHARED((NPAD,), jnp.float32),
            pltpu.SemaphoreType.DMA,
            pltpu.SemaphoreType.DMA,
            pltpu.SemaphoreType.DMA,
        ],
    )(ei, h1d, degp)


def _combine_body(x_ref, b_ref, chn_ref, dinv_ref, acc0_ref, acc1_ref,
                  h1_ref, bg_ref, o_ref):
    i = pl.program_id(0)
    xb = x_ref[...]
    bi = b_ref[...].reshape(BN, 1)
    onehot = (bi == lax.broadcasted_iota(jnp.int32, (BN, G), 1)
              ).astype(jnp.float32)
    chn_rows = jnp.dot(onehot, chn_ref[...], preferred_element_type=jnp.float32)
    sl = pl.ds(i * BN, BN)
    dinv = dinv_ref[sl]
    gcn = dinv * (acc0_ref[sl] + acc1_ref[sl]) + dinv * dinv * h1_ref[sl]
    spa = jax.nn.sigmoid(gcn + bg_ref[...]).reshape(BN, 1)
    o_ref[...] = xb * (1.0 + spa + chn_rows)


def _run_combine(x, batch, chn, dinv, acc0, acc1, h, bg):
    full = pl.BlockSpec((NPAD,), lambda i: (0,))
    return pl.pallas_call(
        _combine_body,
        grid=(GRID,),
        in_specs=[
            pl.BlockSpec((BN, C), lambda i: (i, 0)),
            pl.BlockSpec((BN,), lambda i: (i,)),
            pl.BlockSpec((G, C), lambda i: (0, 0)),
            full, full, full, full,
            pl.BlockSpec((1,), lambda i: (0,)),
        ],
        out_specs=pl.BlockSpec((BN, C), lambda i: (i, 0)),
        out_shape=jax.ShapeDtypeStruct((N, C), jnp.float32),
    )(x, batch, chn, dinv, acc0, acc1, h, bg)


@jax.jit
def kernel(x, batch, edge_index, W1, b1, W2, b2, Wg, bg):
    degp = _run_deg(edge_index)
    h, chn = _run_stats(x, batch, W1, b1, W2, b2, Wg)
    acc0, acc1, dinv = _run_main(edge_index, h, degp)
    return _run_combine(x, batch, chn, dinv, acc0, acc1, h, bg)
